# baseline (speedup 1.0000x reference)
import functools

import jax
import jax.numpy as jnp
from jax import lax
from jax.experimental import pallas as pl
from jax.experimental.pallas import tpu as pltpu, tpu_sc as plsc

F_IN = 128
F_OUT = 128
KSIZE = 9
MULT = 2
N_COLS = KSIZE * F_IN * MULT
N_OUT = 10000

NW = 32
STRIP_ROWS = 40
STRIPS = 256
STRIPS_PER_W = STRIPS // NW
STRIP_WORDS = STRIP_ROWS * N_COLS
CHUNK = 2048
E_PAD_MARGIN = CHUNK + 512


E_REAL = 320000
SEARCH_ROUNDS = 19


def _build_a(dt_p, ids_p, seg_p, seg2d, rate0, rate1):
    mesh = plsc.VectorSubcoreMesh(core_axis_name="c", subcore_axis_name="s")

    @functools.partial(
        pl.kernel,
        out_type=jax.ShapeDtypeStruct((N_OUT * N_COLS,), jnp.float32),
        mesh=mesh,
        compiler_params=pltpu.CompilerParams(needs_layout_passes=False),
        scratch_types=[
            pltpu.VMEM((STRIP_WORDS,), jnp.float32),
            pltpu.VMEM((CHUNK,), jnp.float32),
            pltpu.VMEM((CHUNK,), jnp.int32),
            pltpu.VMEM((CHUNK,), jnp.int32),
            pltpu.VMEM((F_IN,), jnp.float32),
            pltpu.VMEM((F_IN,), jnp.float32),
            pltpu.VMEM((32,), jnp.int32),
            pltpu.VMEM((16,), jnp.int32),
            pltpu.VMEM((16, 128), jnp.int32),
            pltpu.SemaphoreType.DMA,
        ],
    )
    def sc_kernel(dt_hbm, ids_hbm, seg_hbm, seg2d_hbm, r0_hbm, r1_hbm, a_hbm,
                  strip_v, dt_v, ids_v, seg_v, r0_v, r1_v, bnd_v,
                  midr_v, rows_v, sem):
        wid = lax.axis_index("s") * 2 + lax.axis_index("c")
        pltpu.sync_copy(r0_hbm, r0_v)
        pltpu.sync_copy(r1_hbm, r1_v)
        lanes = lax.iota(jnp.int32, 16)

        q = jnp.minimum((wid * STRIPS_PER_W + lanes) * STRIP_ROWS, N_OUT)

        def search_round(_, carry):
            lo, hi = carry
            mid = (lo + hi) >> 1
            midr_v[...] = mid >> 7
            pltpu.async_copy(seg2d_hbm.at[midr_v], rows_v, sem).wait()
            sm = plsc.load_gather(rows_v, [lanes, mid & 127])
            ge = sm >= q
            return (jnp.where(ge, lo, mid + 1), jnp.where(ge, mid, hi))

        lo, hi = lax.fori_loop(
            0, SEARCH_ROUNDS, search_round,
            (jnp.zeros((16,), jnp.int32), jnp.full((16,), E_REAL, jnp.int32)))
        bnd_v[pl.ds(0, 16)] = lo
        bnd_v[pl.ds(16, 16)] = lo

        def do_strip(k, _):
            sid = wid * STRIPS_PER_W + k
            ra = sid * STRIP_ROWS

            @pl.when(ra < N_OUT)
            def _():
                ev = bnd_v[pl.ds(k, 16)]
                estart = ev[0]
                eend = ev[1]

                def zero_body(t, _):
                    base = t * 256
                    for u in range(16):
                        strip_v[pl.ds(base + u * 16, 16)] = jnp.zeros(
                            (16,), jnp.float32)
                    return _
                lax.fori_loop(0, STRIP_WORDS // 256, zero_body, None)

                e0 = (estart // 16) * 16
                nchunks = jnp.maximum(0, (eend - e0 + CHUNK - 1) // CHUNK)

                def chunk_body(c, _):
                    base = e0 + c * CHUNK
                    pltpu.sync_copy(dt_hbm.at[pl.ds(base, CHUNK)], dt_v)
                    pltpu.sync_copy(ids_hbm.at[pl.ds(base, CHUNK)], ids_v)
                    pltpu.sync_copy(seg_hbm.at[pl.ds(base, CHUNK)], seg_v)
                    ng = jnp.minimum(CHUNK, eend - base + 15) // 16

                    def group_body(i, _):
                        off = i * 16
                        dt16 = dt_v[pl.ds(off, 16)]
                        id16 = ids_v[pl.ds(off, 16)]
                        seg16 = seg_v[pl.ds(off, 16)]
                        gidx = base + off + lanes
                        m = (gidx >= estart) & (gidx < eend)
                        c16 = id16 & (F_IN - 1)
                        r0 = plsc.load_gather(r0_v, [c16])
                        r1 = plsc.load_gather(r1_v, [c16])
                        v0 = jnp.exp(-r0 * dt16)
                        v1 = jnp.exp(-r1 * dt16)
                        row = seg16 - ra
                        flat = ((row >> 3) * (18 * 1024)
                                + (id16 >> 6) * 1024
                                + ((row & 7) << 7)
                                + ((id16 & 63) << 1))
                        plsc.addupdate_scatter(strip_v, [flat], v0, mask=m)
                        plsc.addupdate_scatter(strip_v, [flat + 1], v1, mask=m)
                        return _
                    lax.fori_loop(0, ng, group_body, None)
                    return _
                lax.fori_loop(0, nchunks, chunk_body, None)
                pltpu.sync_copy(
                    strip_v, a_hbm.at[pl.ds(ra * N_COLS, STRIP_WORDS)])
            return _
        lax.fori_loop(0, STRIPS_PER_W, do_strip, None)

    return sc_kernel(dt_p, ids_p, seg_p, seg2d, rate0, rate1)


def _matmul_body(bm, a_ref, w_ref, b_ref, o_ref):
    j = pl.program_id(1)
    a2 = a_ref[...].reshape(bm, F_IN)

    @pl.when(j == 0)
    def _():
        o_ref[...] = jnp.zeros((bm, F_OUT), jnp.float32) + b_ref[...]

    o_ref[...] += jnp.dot(a2, w_ref[...][0], preferred_element_type=jnp.float32)


def _matmul(a4, w, bias2d):
    bm = 1000
    n_jt = N_COLS // 128
    return pl.pallas_call(
        functools.partial(_matmul_body, bm),
        grid=(N_OUT // bm, n_jt),
        in_specs=[
            pl.BlockSpec((bm // 8, 1, 8, 128), lambda i, j: (i, j, 0, 0)),
            pl.BlockSpec((1, 128, F_OUT), lambda i, j: (j, 0, 0)),
            pl.BlockSpec((1, F_OUT), lambda i, j: (0, 0)),
        ],
        out_specs=pl.BlockSpec((bm, F_OUT), lambda i, j: (i, 0)),
        out_shape=jax.ShapeDtypeStruct((N_OUT, F_OUT), jnp.float32),
    )(a4, w, bias2d)


def kernel(dt, times_out, successor_kernel_channel_ids, segment_ids_out, decay_rate, kernel, bias):
    e = dt.shape[0]
    e_pad = e + E_PAD_MARGIN
    rate = jax.nn.softplus(decay_rate)
    ids = successor_kernel_channel_ids
    seg = segment_ids_out
    dt_p = jnp.pad(dt, (0, e_pad - e))
    ids_p = jnp.pad(ids, (0, e_pad - e))
    seg_p = jnp.pad(seg, (0, e_pad - e), constant_values=N_OUT)
    seg2d = seg.reshape(e // 128, 128)
    a = _build_a(dt_p, ids_p, seg_p, seg2d,
                 rate[:, 0].copy(), rate[:, 1].copy())
    a4 = a.reshape(N_OUT // 8, KSIZE * MULT, 8, 128)
    w3 = kernel.reshape(N_COLS // 128, 128, F_OUT)
    return _matmul(a4, w3, bias.reshape(1, F_OUT))


# speedup vs baseline: 1.5263x; 1.5263x over previous
import functools

import jax
import jax.numpy as jnp
from jax import lax
from jax.experimental import pallas as pl
from jax.experimental.pallas import tpu as pltpu, tpu_sc as plsc

F_IN = 128
F_OUT = 128
KSIZE = 9
MULT = 2
N_COLS = KSIZE * F_IN * MULT
N_OUT = 10000

NW = 32
STRIP_ROWS = 40
STRIPS = 256
STRIPS_PER_W = STRIPS // NW
STRIP_WORDS = STRIP_ROWS * N_COLS
CHUNK = 2048
E_PAD_MARGIN = CHUNK + 512


E_REAL = 320000
SEARCH_ROUNDS = 19


def _build_a(dt_p, ids_p, seg_p, seg2d, rate0, rate1):
    mesh = plsc.VectorSubcoreMesh(core_axis_name="c", subcore_axis_name="s")

    @functools.partial(
        pl.kernel,
        out_type=jax.ShapeDtypeStruct((N_OUT * N_COLS,), jnp.float32),
        mesh=mesh,
        compiler_params=pltpu.CompilerParams(needs_layout_passes=False),
        scratch_types=[
            pltpu.VMEM((STRIP_WORDS,), jnp.float32),
            pltpu.VMEM((CHUNK,), jnp.float32),
            pltpu.VMEM((CHUNK,), jnp.int32),
            pltpu.VMEM((CHUNK,), jnp.int32),
            pltpu.VMEM((F_IN,), jnp.float32),
            pltpu.VMEM((F_IN,), jnp.float32),
            pltpu.VMEM((32,), jnp.int32),
            pltpu.VMEM((16,), jnp.int32),
            pltpu.VMEM((16, 128), jnp.int32),
            pltpu.SemaphoreType.DMA,
        ],
    )
    def sc_kernel(dt_hbm, ids_hbm, seg_hbm, seg2d_hbm, r0_hbm, r1_hbm, a_hbm,
                  strip_v, dt_v, ids_v, seg_v, r0_v, r1_v, bnd_v,
                  midr_v, rows_v, sem):
        wid = lax.axis_index("s") * 2 + lax.axis_index("c")
        pltpu.sync_copy(r0_hbm, r0_v)
        pltpu.sync_copy(r1_hbm, r1_v)
        lanes = lax.iota(jnp.int32, 16)

        q = jnp.minimum((wid * STRIPS_PER_W + lanes) * STRIP_ROWS, N_OUT)

        def search_round(_, carry):
            lo, hi = carry
            mid = (lo + hi) >> 1
            midr_v[...] = mid >> 7
            pltpu.async_copy(seg2d_hbm.at[midr_v], rows_v, sem).wait()
            sm = plsc.load_gather(rows_v, [lanes, mid & 127])
            ge = sm >= q
            return (jnp.where(ge, lo, mid + 1), jnp.where(ge, mid, hi))

        lo, hi = lax.fori_loop(
            0, SEARCH_ROUNDS, search_round,
            (jnp.zeros((16,), jnp.int32), jnp.full((16,), E_REAL, jnp.int32)))
        bnd_v[pl.ds(0, 16)] = lo
        bnd_v[pl.ds(16, 16)] = lo

        def do_strip(k, _):
            sid = wid * STRIPS_PER_W + k
            ra = sid * STRIP_ROWS

            @pl.when(ra < N_OUT)
            def _():
                ev = bnd_v[pl.ds(k, 16)]
                estart = ev[0]
                eend = ev[1]

                def zero_body(t, _):
                    base = t * 256
                    for u in range(16):
                        strip_v[pl.ds(base + u * 16, 16)] = jnp.zeros(
                            (16,), jnp.float32)
                    return _
                lax.fori_loop(0, STRIP_WORDS // 256, zero_body, None)

                e0 = (estart // 16) * 16
                nchunks = jnp.maximum(0, (eend - e0 + CHUNK - 1) // CHUNK)

                def chunk_body(c, _):
                    base = e0 + c * CHUNK
                    pltpu.sync_copy(dt_hbm.at[pl.ds(base, CHUNK)], dt_v)
                    pltpu.sync_copy(ids_hbm.at[pl.ds(base, CHUNK)], ids_v)
                    pltpu.sync_copy(seg_hbm.at[pl.ds(base, CHUNK)], seg_v)
                    ng = jnp.minimum(CHUNK, eend - base + 15) // 16

                    def group_body(i, _):
                        off = i * 16
                        dt16 = dt_v[pl.ds(off, 16)]
                        id16 = ids_v[pl.ds(off, 16)]
                        seg16 = seg_v[pl.ds(off, 16)]
                        gidx = base + off + lanes
                        m = (gidx >= estart) & (gidx < eend)
                        c16 = id16 & (F_IN - 1)
                        r0 = plsc.load_gather(r0_v, [c16])
                        r1 = plsc.load_gather(r1_v, [c16])
                        v0 = jnp.exp(-r0 * dt16)
                        v1 = jnp.exp(-r1 * dt16)
                        row = seg16 - ra
                        flat = ((id16 >> 6) * (STRIP_ROWS * 128)
                                + (row << 7)
                                + ((id16 & 63) << 1))
                        plsc.addupdate_scatter(strip_v, [flat], v0, mask=m)
                        plsc.addupdate_scatter(strip_v, [flat + 1], v1, mask=m)
                        return _
                    lax.fori_loop(0, ng, group_body, None)
                    return _
                lax.fori_loop(0, nchunks, chunk_body, None)
                descs = []
                for jt in range(N_COLS // 128):
                    descs.append(pltpu.async_copy(
                        strip_v.at[pl.ds(jt * (STRIP_ROWS * 128),
                                         STRIP_ROWS * 128)],
                        a_hbm.at[pl.ds(jt * (N_OUT * 128) + ra * 128,
                                       STRIP_ROWS * 128)],
                        sem))
                for d in descs:
                    d.wait()
            return _
        lax.fori_loop(0, STRIPS_PER_W, do_strip, None)

    return sc_kernel(dt_p, ids_p, seg_p, seg2d, rate0, rate1)


def _matmul_body(bm, a_ref, w_ref, b_ref, o_ref):
    acc = jnp.zeros((bm, F_OUT), jnp.float32) + b_ref[...]
    for jt in range(N_COLS // 128):
        acc = acc + jnp.dot(a_ref[jt], w_ref[jt],
                            preferred_element_type=jnp.float32)
    o_ref[...] = acc


def _matmul(a3, w3, bias2d):
    bm = 1000
    n_jt = N_COLS // 128
    return pl.pallas_call(
        functools.partial(_matmul_body, bm),
        grid=(N_OUT // bm,),
        in_specs=[
            pl.BlockSpec((n_jt, bm, 128), lambda i: (0, i, 0)),
            pl.BlockSpec((n_jt, 128, F_OUT), lambda i: (0, 0, 0)),
            pl.BlockSpec((1, F_OUT), lambda i: (0, 0)),
        ],
        out_specs=pl.BlockSpec((bm, F_OUT), lambda i: (i, 0)),
        out_shape=jax.ShapeDtypeStruct((N_OUT, F_OUT), jnp.float32),
    )(a3, w3, bias2d)


def kernel(dt, times_out, successor_kernel_channel_ids, segment_ids_out, decay_rate, kernel, bias):
    e = dt.shape[0]
    e_pad = e + E_PAD_MARGIN
    rate = jax.nn.softplus(decay_rate)
    ids = successor_kernel_channel_ids
    seg = segment_ids_out
    dt_p = jnp.pad(dt, (0, e_pad - e))
    ids_p = jnp.pad(ids, (0, e_pad - e))
    seg_p = jnp.pad(seg, (0, e_pad - e), constant_values=N_OUT)
    seg2d = seg.reshape(e // 128, 128)
    a = _build_a(dt_p, ids_p, seg_p, seg2d,
                 rate[:, 0].copy(), rate[:, 1].copy())
    a3 = a.reshape(N_COLS // 128, N_OUT, 128)
    w3 = kernel.reshape(N_COLS // 128, 128, F_OUT)
    return _matmul(a3, w3, bias.reshape(1, F_OUT))
